# baseline (speedup 1.0000x reference)
import jax
import jax.numpy as jnp
from jax import lax
from jax.experimental import pallas as pl
from jax.experimental.pallas import tpu as pltpu
from jax.experimental.pallas import tpu_sc as plsc

N = 10000
D = 128
H = 256
E = 320000

NC = 2
NS = 16
B = 80
NB = E // NS // B
IB = 50
NIB = NB // IB
NPAD = 10240
CROWS = NPAD // NS
CB = 125

_MESH = plsc.VectorSubcoreMesh(core_axis_name="c", subcore_axis_name="s")


def _agg_body(x2n, src_r, dst_r, zeros, sums,
              srcv, dstv, buf0, buf1, acc, sem0, sem1, ssem0, ssem1):
    c = lax.axis_index("c")
    s = lax.axis_index("s")

    pltpu.sync_copy(zeros, buf0)
    base = s * CROWS
    for i in range(CROWS // B):
        pltpu.sync_copy(buf0, acc.at[pl.ds(base + i * B, B)])
    plsc.subcore_barrier()

    bufs = (buf0, buf1)
    sems = (sem0, sem1)
    ssems = (ssem0, ssem1)

    def gstart(j, k):
        pltpu.async_copy(x2n.at[srcv.at[j]], bufs[k], sems[k])

    def gwait(k):
        pltpu.make_async_copy(x2n.at[pl.ds(0, B)], bufs[k], sems[k]).wait()

    def sstart(j, k):
        pltpu.async_copy(bufs[k], acc.at[dstv.at[j]], ssems[k], add=True)

    def swait(k):
        pltpu.make_async_copy(bufs[k], acc.at[pl.ds(0, B)], ssems[k]).wait()

    def chunk(g, carry):
        pltpu.sync_copy(src_r.at[c, s, g], srcv)
        pltpu.sync_copy(dst_r.at[s, g], dstv)
        gstart(0, 0)
        gstart(1, 1)

        def step(t, cy):
            j = 2 * t
            gwait(0)
            sstart(j, 0)
            gwait(1)
            sstart(j + 1, 1)
            swait(0)
            gstart(j + 2, 0)
            swait(1)
            gstart(j + 3, 1)
            return cy

        lax.fori_loop(0, IB // 2 - 1, step, 0)
        gwait(0)
        sstart(IB - 2, 0)
        gwait(1)
        sstart(IB - 1, 1)
        swait(0)
        swait(1)
        return carry

    lax.fori_loop(0, NIB, chunk, 0)
    plsc.subcore_barrier()

    def drain(nchunks):
        for i in range(nchunks):
            r = base + i * B
            pltpu.sync_copy(acc.at[pl.ds(r, B)], buf0)
            pltpu.sync_copy(buf0, sums.at[pl.ds(c * N + r, B)])

    @pl.when(s < NS - 1)
    def _():
        drain(CROWS // B)

    @pl.when(s == NS - 1)
    def _():
        drain((N - (NS - 1) * CROWS) // B)


_agg = pl.kernel(
    _agg_body,
    out_type=jax.ShapeDtypeStruct((2 * N, D), jnp.float32),
    mesh=_MESH,
    scratch_types=[
        pltpu.VMEM((IB, B), jnp.int32),
        pltpu.VMEM((IB, B), jnp.int32),
        pltpu.VMEM((B, D), jnp.float32),
        pltpu.VMEM((B, D), jnp.float32),
        pltpu.VMEM_SHARED((NPAD, D), jnp.float32),
        pltpu.SemaphoreType.DMA,
        pltpu.SemaphoreType.DMA,
        pltpu.SemaphoreType.DMA,
        pltpu.SemaphoreType.DMA,
    ],
)


def _cnt_body(dst_r, zeros, ones, cnt_out, dstv, zb, onesb, cacc):
    c = lax.axis_index("c")
    s = lax.axis_index("s")
    wid = c * NS + s

    pltpu.sync_copy(dst_r.at[wid], dstv)
    pltpu.sync_copy(zeros, zb)
    pltpu.sync_copy(ones, onesb)
    base = s * CROWS
    for i in range(CROWS // B):
        pltpu.sync_copy(zb, cacc.at[pl.ds(base + i * B, B)])
    plsc.subcore_barrier()

    def scat(j, carry):
        pltpu.sync_copy(onesb, cacc.at[dstv.at[j]], add=True)
        return carry

    lax.fori_loop(0, CB, scat, 0)
    plsc.subcore_barrier()

    for i in range(CROWS // B):
        r = base + i * B
        pltpu.sync_copy(cacc.at[pl.ds(r, B)], zb)
        pltpu.sync_copy(zb, cnt_out.at[c, pl.ds(r, B)])


_cnt = pl.kernel(
    _cnt_body,
    out_type=jax.ShapeDtypeStruct((NC, NPAD, D), jnp.float32),
    mesh=_MESH,
    scratch_types=[
        pltpu.VMEM((CB, B), jnp.int32),
        pltpu.VMEM((B, D), jnp.float32),
        pltpu.VMEM((B, D), jnp.float32),
        pltpu.VMEM_SHARED((NPAD, D), jnp.float32),
    ],
)


def _layer_tc(s2n, cnta, cntb, W, b, relu):
    BR = 400

    def body(x0_ref, x1_ref, ca_ref, cb_ref, w_ref, b_ref, o_ref):
        cnt = ca_ref[...] + cb_ref[...]
        r = 1.0 / jnp.maximum(cnt, 1.0)
        x0 = x0_ref[...] * r
        x1 = x1_ref[...] * r
        h = jnp.dot(x0, w_ref[0:D, :], preferred_element_type=jnp.float32)
        h = h + jnp.dot(x1, w_ref[D:H, :], preferred_element_type=jnp.float32)
        h = h + b_ref[...]
        if relu:
            h = jnp.maximum(h, 0.0)
        o_ref[...] = h

    nb = N // BR
    return pl.pallas_call(
        body,
        grid=(2, nb),
        in_specs=[
            pl.BlockSpec((BR, D), lambda g, i: (i, 0)),
            pl.BlockSpec((BR, D), lambda g, i: (i + nb, 0)),
            pl.BlockSpec((BR, 1), lambda g, i: (i, 0)),
            pl.BlockSpec((BR, 1), lambda g, i: (i, 0)),
            pl.BlockSpec((H, D), lambda g, i: (0, g)),
            pl.BlockSpec((1, D), lambda g, i: (0, g)),
        ],
        out_specs=pl.BlockSpec((BR, D), lambda g, i: (g * nb + i, 0)),
        out_shape=jax.ShapeDtypeStruct((2 * N, D), jnp.float32),
    )(s2n, s2n, cnta, cntb, W, b.reshape(1, H))


def _head_tc(hi, hs, Wpi, bpi, Wps, bps, Wf, bf):
    BR = 400
    nb = N // BR

    def body(hi0_ref, hi1_ref, hs0_ref, hs1_ref,
             wpi_ref, bpi_ref, wps_ref, bps_ref, wf_ref, bf_ref,
             h_ref, zi_ref, zs_ref):
        hi0 = hi0_ref[...]
        hi1 = hi1_ref[...]
        hs0 = hs0_ref[...]
        hs1 = hs1_ref[...]
        zi = jnp.dot(hi0, wpi_ref[0:D, :], preferred_element_type=jnp.float32)
        zi = zi + jnp.dot(hi1, wpi_ref[D:H, :], preferred_element_type=jnp.float32)
        zi_ref[...] = jnp.maximum(zi + bpi_ref[...], 0.0)
        zs = jnp.dot(hs0, wps_ref[0:D, :], preferred_element_type=jnp.float32)
        zs = zs + jnp.dot(hs1, wps_ref[D:H, :], preferred_element_type=jnp.float32)
        zs_ref[...] = jnp.maximum(zs + bps_ref[...], 0.0)
        h = jnp.dot(hi0 + hs0, wf_ref[0:D, :], preferred_element_type=jnp.float32)
        h = h + jnp.dot(hi1 + hs1, wf_ref[D:H, :], preferred_element_type=jnp.float32)
        h_ref[...] = h + bf_ref[...]

    return pl.pallas_call(
        body,
        grid=(nb,),
        in_specs=[
            pl.BlockSpec((BR, D), lambda i: (i, 0)),
            pl.BlockSpec((BR, D), lambda i: (i + nb, 0)),
            pl.BlockSpec((BR, D), lambda i: (i, 0)),
            pl.BlockSpec((BR, D), lambda i: (i + nb, 0)),
            pl.BlockSpec((H, H), lambda i: (0, 0)),
            pl.BlockSpec((1, H), lambda i: (0, 0)),
            pl.BlockSpec((H, H), lambda i: (0, 0)),
            pl.BlockSpec((1, H), lambda i: (0, 0)),
            pl.BlockSpec((H, D), lambda i: (0, 0)),
            pl.BlockSpec((1, D), lambda i: (0, 0)),
        ],
        out_specs=[
            pl.BlockSpec((BR, D), lambda i: (i, 0)),
            pl.BlockSpec((BR, H), lambda i: (i, 0)),
            pl.BlockSpec((BR, H), lambda i: (i, 0)),
        ],
        out_shape=[
            jax.ShapeDtypeStruct((N, D), jnp.float32),
            jax.ShapeDtypeStruct((N, H), jnp.float32),
            jax.ShapeDtypeStruct((N, H), jnp.float32),
        ],
    )(hi, hi, hs, hs, Wpi, bpi.reshape(1, H), Wps, bps.reshape(1, H),
      Wf, bf.reshape(1, D))


def kernel(x, x_sim, edge_index_inter, edge_index_sim,
           W1i, b1i, W2i, b2i, W3i, b3i,
           W1s, b1s, W2s, b2s, W3s, b3s,
           Wpi, bpi, Wps, bps, Wf, bf):
    x2n = jnp.concatenate([x, x_sim], axis=0)
    zeros = jnp.zeros((B, D), jnp.float32)
    ones = jnp.ones((B, D), jnp.float32)

    def stack(edge_index, Ws, bs):
        src = edge_index[0].reshape(NS, NIB, IB, B)
        dst = edge_index[1].reshape(NS, NIB, IB, B)
        src_r = jnp.stack([src, src + N])
        dst_c = edge_index[1].reshape(NC * NS, CB, B)

        cnt = _cnt(dst_c, zeros, ones)
        cnta = cnt[0, :N, 0:1]
        cntb = cnt[1, :N, 0:1]
        sums = _agg(x2n, src_r, dst, zeros)
        h = _layer_tc(sums, cnta, cntb, Ws[0], bs[0], relu=True)
        sums = _agg(h, src_r, dst, zeros)
        h = _layer_tc(sums, cnta, cntb, Ws[1], bs[1], relu=True)
        sums = _agg(h, src_r, dst, zeros)
        h = _layer_tc(sums, cnta, cntb, Ws[2], bs[2], relu=False)
        return h

    hi = stack(edge_index_inter, (W1i, W2i, W3i), (b1i, b2i, b3i))
    hs = stack(edge_index_sim, (W1s, W2s, W3s), (b1s, b2s, b3s))
    return _head_tc(hi, hs, Wpi, bpi, Wps, bps, Wf, bf)


# speedup vs baseline: 1.3726x; 1.3726x over previous
import jax
import jax.numpy as jnp
from jax import lax
from jax.experimental import pallas as pl
from jax.experimental.pallas import tpu as pltpu
from jax.experimental.pallas import tpu_sc as plsc

N = 10000
D = 128
H = 256
E = 320000

NC = 2
NS = 16
B = 80
NB = E // NS // B
IB = 25
NIB = NB // IB
NBUF = 3
NPAD = 10240
CROWS = NPAD // NS
CB = 125

_MESH = plsc.VectorSubcoreMesh(core_axis_name="c", subcore_axis_name="s")


def _agg_body(x2n, src_r, dst_r, zeros, sums,
              srcv, dstv, buf0, buf1, buf2, acc, sem0, sem1, sem2):
    c = lax.axis_index("c")
    s = lax.axis_index("s")

    pltpu.sync_copy(zeros, buf0)
    base = s * CROWS
    for i in range(CROWS // B):
        pltpu.sync_copy(buf0, acc.at[pl.ds(base + i * B, B)])
    plsc.subcore_barrier()

    bufs = (buf0, buf1, buf2)
    sems = (sem0, sem1, sem2)

    def gstart(j, k):
        pltpu.async_copy(x2n.at[srcv.at[j]], bufs[k], sems[k])

    def gwait(k):
        pltpu.make_async_copy(x2n.at[pl.ds(0, B)], bufs[k], sems[k]).wait()

    def scat(j, k):
        pltpu.sync_copy(bufs[k], acc.at[dstv.at[j]], add=True)

    def chunk(g, carry):
        pltpu.sync_copy(src_r.at[c, s, g], srcv)
        pltpu.sync_copy(dst_r.at[s, g], dstv)
        for k in range(NBUF):
            gstart(k, k)

        def step(t, cy):
            j = NBUF * t
            for k in range(NBUF):
                gwait(k)
                scat(j + k, k)
                gstart(j + NBUF + k, k)
            return cy

        lax.fori_loop(0, (IB - 1) // NBUF - 1, step, 0)
        gwait(0)
        scat(IB - 4, 0)
        gstart(IB - 1, 0)
        gwait(1)
        scat(IB - 3, 1)
        gwait(2)
        scat(IB - 2, 2)
        gwait(0)
        scat(IB - 1, 0)
        return carry

    lax.fori_loop(0, NIB, chunk, 0)
    plsc.subcore_barrier()

    def drain(nchunks):
        for i in range(nchunks):
            r = base + i * B
            pltpu.sync_copy(acc.at[pl.ds(r, B)], buf0)
            pltpu.sync_copy(buf0, sums.at[pl.ds(c * N + r, B)])

    @pl.when(s < NS - 1)
    def _():
        drain(CROWS // B)

    @pl.when(s == NS - 1)
    def _():
        drain((N - (NS - 1) * CROWS) // B)


_agg = pl.kernel(
    _agg_body,
    out_type=jax.ShapeDtypeStruct((2 * N, D), jnp.float32),
    mesh=_MESH,
    scratch_types=[
        pltpu.VMEM((IB, B), jnp.int32),
        pltpu.VMEM((IB, B), jnp.int32),
        pltpu.VMEM((B, D), jnp.float32),
        pltpu.VMEM((B, D), jnp.float32),
        pltpu.VMEM((B, D), jnp.float32),
        pltpu.VMEM_SHARED((NPAD, D), jnp.float32),
        pltpu.SemaphoreType.DMA,
        pltpu.SemaphoreType.DMA,
        pltpu.SemaphoreType.DMA,
    ],
)


def _cnt_body(dst_r, zeros, ones, cnt_out, dstv, zb, onesb, cacc):
    c = lax.axis_index("c")
    s = lax.axis_index("s")
    wid = c * NS + s

    pltpu.sync_copy(dst_r.at[wid], dstv)
    pltpu.sync_copy(zeros, zb)
    pltpu.sync_copy(ones, onesb)
    base = s * CROWS
    for i in range(CROWS // B):
        pltpu.sync_copy(zb, cacc.at[pl.ds(base + i * B, B)])
    plsc.subcore_barrier()

    def scat(j, carry):
        pltpu.sync_copy(onesb, cacc.at[dstv.at[j]], add=True)
        return carry

    lax.fori_loop(0, CB, scat, 0)
    plsc.subcore_barrier()

    for i in range(CROWS // B):
        r = base + i * B
        pltpu.sync_copy(cacc.at[pl.ds(r, B)], zb)
        pltpu.sync_copy(zb, cnt_out.at[c, pl.ds(r, B)])


_cnt = pl.kernel(
    _cnt_body,
    out_type=jax.ShapeDtypeStruct((NC, NPAD, D), jnp.float32),
    mesh=_MESH,
    scratch_types=[
        pltpu.VMEM((CB, B), jnp.int32),
        pltpu.VMEM((B, D), jnp.float32),
        pltpu.VMEM((B, D), jnp.float32),
        pltpu.VMEM_SHARED((NPAD, D), jnp.float32),
    ],
)


def _layer_tc(s2n, cnta, cntb, W, b, relu):
    BR = 400

    def body(x0_ref, x1_ref, ca_ref, cb_ref, w_ref, b_ref, o_ref):
        cnt = ca_ref[...] + cb_ref[...]
        r = 1.0 / jnp.maximum(cnt, 1.0)
        x0 = x0_ref[...] * r
        x1 = x1_ref[...] * r
        h = jnp.dot(x0, w_ref[0:D, :], preferred_element_type=jnp.float32)
        h = h + jnp.dot(x1, w_ref[D:H, :], preferred_element_type=jnp.float32)
        h = h + b_ref[...]
        if relu:
            h = jnp.maximum(h, 0.0)
        o_ref[...] = h

    nb = N // BR
    return pl.pallas_call(
        body,
        grid=(2, nb),
        in_specs=[
            pl.BlockSpec((BR, D), lambda g, i: (i, 0)),
            pl.BlockSpec((BR, D), lambda g, i: (i + nb, 0)),
            pl.BlockSpec((BR, 1), lambda g, i: (i, 0)),
            pl.BlockSpec((BR, 1), lambda g, i: (i, 0)),
            pl.BlockSpec((H, D), lambda g, i: (0, g)),
            pl.BlockSpec((1, D), lambda g, i: (0, g)),
        ],
        out_specs=pl.BlockSpec((BR, D), lambda g, i: (g * nb + i, 0)),
        out_shape=jax.ShapeDtypeStruct((2 * N, D), jnp.float32),
    )(s2n, s2n, cnta, cntb, W, b.reshape(1, H))


def _head_tc(hi, hs, Wpi, bpi, Wps, bps, Wf, bf):
    BR = 400
    nb = N // BR

    def body(hi0_ref, hi1_ref, hs0_ref, hs1_ref,
             wpi_ref, bpi_ref, wps_ref, bps_ref, wf_ref, bf_ref,
             h_ref, zi_ref, zs_ref):
        hi0 = hi0_ref[...]
        hi1 = hi1_ref[...]
        hs0 = hs0_ref[...]
        hs1 = hs1_ref[...]
        zi = jnp.dot(hi0, wpi_ref[0:D, :], preferred_element_type=jnp.float32)
        zi = zi + jnp.dot(hi1, wpi_ref[D:H, :], preferred_element_type=jnp.float32)
        zi_ref[...] = jnp.maximum(zi + bpi_ref[...], 0.0)
        zs = jnp.dot(hs0, wps_ref[0:D, :], preferred_element_type=jnp.float32)
        zs = zs + jnp.dot(hs1, wps_ref[D:H, :], preferred_element_type=jnp.float32)
        zs_ref[...] = jnp.maximum(zs + bps_ref[...], 0.0)
        h = jnp.dot(hi0 + hs0, wf_ref[0:D, :], preferred_element_type=jnp.float32)
        h = h + jnp.dot(hi1 + hs1, wf_ref[D:H, :], preferred_element_type=jnp.float32)
        h_ref[...] = h + bf_ref[...]

    return pl.pallas_call(
        body,
        grid=(nb,),
        in_specs=[
            pl.BlockSpec((BR, D), lambda i: (i, 0)),
            pl.BlockSpec((BR, D), lambda i: (i + nb, 0)),
            pl.BlockSpec((BR, D), lambda i: (i, 0)),
            pl.BlockSpec((BR, D), lambda i: (i + nb, 0)),
            pl.BlockSpec((H, H), lambda i: (0, 0)),
            pl.BlockSpec((1, H), lambda i: (0, 0)),
            pl.BlockSpec((H, H), lambda i: (0, 0)),
            pl.BlockSpec((1, H), lambda i: (0, 0)),
            pl.BlockSpec((H, D), lambda i: (0, 0)),
            pl.BlockSpec((1, D), lambda i: (0, 0)),
        ],
        out_specs=[
            pl.BlockSpec((BR, D), lambda i: (i, 0)),
            pl.BlockSpec((BR, H), lambda i: (i, 0)),
            pl.BlockSpec((BR, H), lambda i: (i, 0)),
        ],
        out_shape=[
            jax.ShapeDtypeStruct((N, D), jnp.float32),
            jax.ShapeDtypeStruct((N, H), jnp.float32),
            jax.ShapeDtypeStruct((N, H), jnp.float32),
        ],
    )(hi, hi, hs, hs, Wpi, bpi.reshape(1, H), Wps, bps.reshape(1, H),
      Wf, bf.reshape(1, D))


def kernel(x, x_sim, edge_index_inter, edge_index_sim,
           W1i, b1i, W2i, b2i, W3i, b3i,
           W1s, b1s, W2s, b2s, W3s, b3s,
           Wpi, bpi, Wps, bps, Wf, bf):
    x2n = jnp.concatenate([x, x_sim], axis=0)
    zeros = jnp.zeros((B, D), jnp.float32)
    ones = jnp.ones((B, D), jnp.float32)

    def stack(edge_index, Ws, bs):
        src = edge_index[0].reshape(NS, NIB, IB, B)
        dst = edge_index[1].reshape(NS, NIB, IB, B)
        src_r = jnp.stack([src, src + N])
        dst_c = edge_index[1].reshape(NC * NS, CB, B)

        cnt = _cnt(dst_c, zeros, ones)
        cnta = cnt[0, :N, 0:1]
        cntb = cnt[1, :N, 0:1]
        sums = _agg(x2n, src_r, dst, zeros)
        h = _layer_tc(sums, cnta, cntb, Ws[0], bs[0], relu=True)
        sums = _agg(h, src_r, dst, zeros)
        h = _layer_tc(sums, cnta, cntb, Ws[1], bs[1], relu=True)
        sums = _agg(h, src_r, dst, zeros)
        h = _layer_tc(sums, cnta, cntb, Ws[2], bs[2], relu=False)
        return h

    hi = stack(edge_index_inter, (W1i, W2i, W3i), (b1i, b2i, b3i))
    hs = stack(edge_index_sim, (W1s, W2s, W3s), (b1s, b2s, b3s))
    return _head_tc(hi, hs, Wpi, bpi, Wps, bps, Wf, bf)


# speedup vs baseline: 1.3727x; 1.0001x over previous
import jax
import jax.numpy as jnp
from jax import lax
from jax.experimental import pallas as pl
from jax.experimental.pallas import tpu as pltpu
from jax.experimental.pallas import tpu_sc as plsc

N = 10000
D = 128
H = 256
E = 320000

NC = 2
NS = 16
B = 80
NB = E // NS // B
IB = 25
NIB = NB // IB
NBUF = 3
NPAD = 10240
CROWS = NPAD // NS
CB = 125

_MESH = plsc.VectorSubcoreMesh(core_axis_name="c", subcore_axis_name="s")


def _agg_body(x2n, src_r, dst_r, zeros, sums,
              srcv, dstv, buf0, buf1, buf2, acc, sem0, sem1, sem2):
    c = lax.axis_index("c")
    s = lax.axis_index("s")

    pltpu.sync_copy(zeros, buf0)
    base = s * CROWS
    for i in range(CROWS // B):
        pltpu.sync_copy(buf0, acc.at[pl.ds(base + i * B, B)])
    plsc.subcore_barrier()

    bufs = (buf0, buf1, buf2)
    sems = (sem0, sem1, sem2)

    def gstart(j, k):
        pltpu.async_copy(x2n.at[srcv.at[j]], bufs[k], sems[k])

    def gwait(k):
        pltpu.make_async_copy(x2n.at[pl.ds(0, B)], bufs[k], sems[k]).wait()

    def scat(j, k):
        pltpu.sync_copy(bufs[k], acc.at[dstv.at[j]], add=True)

    def chunk(g, carry):
        pltpu.sync_copy(src_r.at[c, s, g], srcv)
        pltpu.sync_copy(dst_r.at[s, g], dstv)
        for k in range(NBUF):
            gstart(k, k)

        def step(t, cy):
            j = NBUF * t
            for k in range(NBUF):
                gwait(k)
                scat(j + k, k)
                gstart(j + NBUF + k, k)
            return cy

        lax.fori_loop(0, (IB - 1) // NBUF - 1, step, 0)
        gwait(0)
        scat(IB - 4, 0)
        gstart(IB - 1, 0)
        gwait(1)
        scat(IB - 3, 1)
        gwait(2)
        scat(IB - 2, 2)
        gwait(0)
        scat(IB - 1, 0)
        return carry

    lax.fori_loop(0, NIB, chunk, 0)
    plsc.subcore_barrier()

    def drain(nchunks):
        for i in range(nchunks):
            r = base + i * B
            pltpu.sync_copy(acc.at[pl.ds(r, B)], buf0)
            pltpu.sync_copy(buf0, sums.at[pl.ds(c * N + r, B)])

    @pl.when(s < NS - 1)
    def _():
        drain(CROWS // B)

    @pl.when(s == NS - 1)
    def _():
        drain((N - (NS - 1) * CROWS) // B)


_agg = pl.kernel(
    _agg_body,
    out_type=jax.ShapeDtypeStruct((2 * N, D), jnp.float32),
    mesh=_MESH,
    scratch_types=[
        pltpu.VMEM((IB, B), jnp.int32),
        pltpu.VMEM((IB, B), jnp.int32),
        pltpu.VMEM((B, D), jnp.float32),
        pltpu.VMEM((B, D), jnp.float32),
        pltpu.VMEM((B, D), jnp.float32),
        pltpu.VMEM_SHARED((NPAD, D), jnp.float32),
        pltpu.SemaphoreType.DMA,
        pltpu.SemaphoreType.DMA,
        pltpu.SemaphoreType.DMA,
    ],
)


def _cnt_body(dst_r, zeros, ones, cnt_out, dstv, zb, onesb, cacc):
    c = lax.axis_index("c")
    s = lax.axis_index("s")
    wid = c * NS + s

    pltpu.sync_copy(dst_r.at[wid], dstv)
    pltpu.sync_copy(zeros, zb)
    pltpu.sync_copy(ones, onesb)
    base = s * CROWS
    for i in range(CROWS // B):
        pltpu.sync_copy(zb, cacc.at[pl.ds(base + i * B, B)])
    plsc.subcore_barrier()

    def scat(j, carry):
        pltpu.sync_copy(onesb, cacc.at[dstv.at[j]], add=True)
        return carry

    lax.fori_loop(0, CB, scat, 0)
    plsc.subcore_barrier()

    for i in range(CROWS // B):
        r = base + i * B
        pltpu.sync_copy(cacc.at[pl.ds(r, B)], zb)
        pltpu.sync_copy(zb, cnt_out.at[c, pl.ds(r, B)])


_cnt = pl.kernel(
    _cnt_body,
    out_type=jax.ShapeDtypeStruct((NC, NPAD, D), jnp.float32),
    mesh=_MESH,
    scratch_types=[
        pltpu.VMEM((CB, B), jnp.int32),
        pltpu.VMEM((B, D), jnp.float32),
        pltpu.VMEM((B, D), jnp.float32),
        pltpu.VMEM_SHARED((NPAD, D), jnp.float32),
    ],
)


def _layer_tc(s2n, cnta, cntb, W, b, relu):
    BR = 400

    def body(x0_ref, x1_ref, ca_ref, cb_ref, w_ref, b_ref, o_ref):
        cnt = ca_ref[...] + cb_ref[...]
        r = 1.0 / jnp.maximum(cnt, 1.0)
        x0 = x0_ref[...] * r
        x1 = x1_ref[...] * r
        h = jnp.dot(x0, w_ref[0:D, :], preferred_element_type=jnp.float32)
        h = h + jnp.dot(x1, w_ref[D:H, :], preferred_element_type=jnp.float32)
        h = h + b_ref[...]
        if relu:
            h = jnp.maximum(h, 0.0)
        o_ref[...] = h

    nb = N // BR
    return pl.pallas_call(
        body,
        grid=(2, nb),
        in_specs=[
            pl.BlockSpec((BR, D), lambda g, i: (i, 0)),
            pl.BlockSpec((BR, D), lambda g, i: (i + nb, 0)),
            pl.BlockSpec((BR, 1), lambda g, i: (i, 0)),
            pl.BlockSpec((BR, 1), lambda g, i: (i, 0)),
            pl.BlockSpec((H, D), lambda g, i: (0, g)),
            pl.BlockSpec((1, D), lambda g, i: (0, g)),
        ],
        out_specs=pl.BlockSpec((BR, D), lambda g, i: (g * nb + i, 0)),
        out_shape=jax.ShapeDtypeStruct((2 * N, D), jnp.float32),
    )(s2n, s2n, cnta, cntb, W, b.reshape(1, H))


def _head_tc(hi, hs, Wpi, bpi, Wps, bps, Wf, bf):
    BR = 400
    nb = N // BR

    def body(hi0_ref, hi1_ref, hs0_ref, hs1_ref,
             wpi_ref, bpi_ref, wps_ref, bps_ref, wf_ref, bf_ref,
             h_ref, zi_ref, zs_ref):
        hi0 = hi0_ref[...]
        hi1 = hi1_ref[...]
        hs0 = hs0_ref[...]
        hs1 = hs1_ref[...]
        zi = jnp.dot(hi0, wpi_ref[0:D, :], preferred_element_type=jnp.float32)
        zi = zi + jnp.dot(hi1, wpi_ref[D:H, :], preferred_element_type=jnp.float32)
        zi_ref[...] = jnp.maximum(zi + bpi_ref[...], 0.0)
        zs = jnp.dot(hs0, wps_ref[0:D, :], preferred_element_type=jnp.float32)
        zs = zs + jnp.dot(hs1, wps_ref[D:H, :], preferred_element_type=jnp.float32)
        zs_ref[...] = jnp.maximum(zs + bps_ref[...], 0.0)
        h = jnp.dot(hi0 + hs0, wf_ref[0:D, :], preferred_element_type=jnp.float32)
        h = h + jnp.dot(hi1 + hs1, wf_ref[D:H, :], preferred_element_type=jnp.float32)
        h_ref[...] = h + bf_ref[...]

    return pl.pallas_call(
        body,
        grid=(nb,),
        in_specs=[
            pl.BlockSpec((BR, D), lambda i: (i, 0)),
            pl.BlockSpec((BR, D), lambda i: (i + nb, 0)),
            pl.BlockSpec((BR, D), lambda i: (i, 0)),
            pl.BlockSpec((BR, D), lambda i: (i + nb, 0)),
            pl.BlockSpec((H, H), lambda i: (0, 0)),
            pl.BlockSpec((1, H), lambda i: (0, 0)),
            pl.BlockSpec((H, H), lambda i: (0, 0)),
            pl.BlockSpec((1, H), lambda i: (0, 0)),
            pl.BlockSpec((H, D), lambda i: (0, 0)),
            pl.BlockSpec((1, D), lambda i: (0, 0)),
        ],
        out_specs=[
            pl.BlockSpec((BR, D), lambda i: (i, 0)),
            pl.BlockSpec((BR, H), lambda i: (i, 0)),
            pl.BlockSpec((BR, H), lambda i: (i, 0)),
        ],
        out_shape=[
            jax.ShapeDtypeStruct((N, D), jnp.float32),
            jax.ShapeDtypeStruct((N, H), jnp.float32),
            jax.ShapeDtypeStruct((N, H), jnp.float32),
        ],
    )(hi, hi, hs, hs, Wpi, bpi.reshape(1, H), Wps, bps.reshape(1, H),
      Wf, bf.reshape(1, D))


def kernel(x, x_sim, edge_index_inter, edge_index_sim,
           W1i, b1i, W2i, b2i, W3i, b3i,
           W1s, b1s, W2s, b2s, W3s, b3s,
           Wpi, bpi, Wps, bps, Wf, bf):
    x2n = jnp.concatenate([x, x_sim], axis=0)
    zeros = jnp.zeros((B, D), jnp.float32)
    ones = jnp.ones((B, D), jnp.float32)

    def prep(edge_index):
        src = edge_index[0].reshape(NS, NIB, IB, B)
        dst = edge_index[1].reshape(NS, NIB, IB, B)
        src_r = jnp.stack([src, src + N])
        dst_c = edge_index[1].reshape(NC * NS, CB, B)
        cnt = _cnt(dst_c, zeros, ones)
        return src_r, dst, cnt[0, :N, 0:1], cnt[1, :N, 0:1]

    gi = prep(edge_index_inter)
    gs = prep(edge_index_sim)
    Wi = ((W1i, b1i), (W2i, b2i), (W3i, b3i))
    Ws = ((W1s, b1s), (W2s, b2s), (W3s, b3s))
    hi, hs = x2n, x2n
    for l in range(3):
        relu = l < 2
        si = _agg(hi, gi[0], gi[1], zeros)
        ss = _agg(hs, gs[0], gs[1], zeros)
        hi = _layer_tc(si, gi[2], gi[3], Wi[l][0], Wi[l][1], relu=relu)
        hs = _layer_tc(ss, gs[2], gs[3], Ws[l][0], Ws[l][1], relu=relu)
    return _head_tc(hi, hs, Wpi, bpi, Wps, bps, Wf, bf)


# speedup vs baseline: 1.3877x; 1.0109x over previous
import jax
import jax.numpy as jnp
from jax import lax
from jax.experimental import pallas as pl
from jax.experimental.pallas import tpu as pltpu
from jax.experimental.pallas import tpu_sc as plsc

N = 10000
D = 128
H = 256
E = 320000

NC = 2
NS = 16
B = 80
NB = E // NS // B
IB = 25
NIB = NB // IB
NBUF = 3
NPAD = 10240
CROWS = NPAD // NS
CB = 125

_MESH = plsc.VectorSubcoreMesh(core_axis_name="c", subcore_axis_name="s")


def _agg_body(x2n, src_r, dst_r, zeros, sums,
              srcv, dstv, buf0, buf1, buf2, acc, sem0, sem1, sem2):
    c = lax.axis_index("c")
    s = lax.axis_index("s")

    pltpu.sync_copy(zeros, buf0)
    base = s * CROWS
    for i in range(CROWS // B):
        pltpu.async_copy(buf0, acc.at[pl.ds(base + i * B, B)], sem0)
    for i in range(CROWS // B):
        pltpu.make_async_copy(buf0, acc.at[pl.ds(base, B)], sem0).wait()
    plsc.subcore_barrier()

    bufs = (buf0, buf1, buf2)
    sems = (sem0, sem1, sem2)

    def gstart(j, k):
        pltpu.async_copy(x2n.at[srcv.at[j]], bufs[k], sems[k])

    def gwait(k):
        pltpu.make_async_copy(x2n.at[pl.ds(0, B)], bufs[k], sems[k]).wait()

    def scat(j, k):
        pltpu.sync_copy(bufs[k], acc.at[dstv.at[j]], add=True)

    def chunk(g, carry):
        pltpu.sync_copy(src_r.at[c, s, g], srcv)
        pltpu.sync_copy(dst_r.at[s, g], dstv)
        for k in range(NBUF):
            gstart(k, k)

        def step(t, cy):
            j = NBUF * t
            for k in range(NBUF):
                gwait(k)
                scat(j + k, k)
                gstart(j + NBUF + k, k)
            return cy

        lax.fori_loop(0, (IB - 1) // NBUF - 1, step, 0)
        gwait(0)
        scat(IB - 4, 0)
        gstart(IB - 1, 0)
        gwait(1)
        scat(IB - 3, 1)
        gwait(2)
        scat(IB - 2, 2)
        gwait(0)
        scat(IB - 1, 0)
        return carry

    lax.fori_loop(0, NIB, chunk, 0)
    plsc.subcore_barrier()

    def drain(nchunks):
        for i in range(nchunks):
            r = base + i * B
            k = i % NBUF
            if i >= NBUF:
                pltpu.make_async_copy(bufs[k], sums.at[pl.ds(0, B)],
                                      sems[k]).wait()
            pltpu.sync_copy(acc.at[pl.ds(r, B)], bufs[k])
            pltpu.async_copy(bufs[k], sums.at[pl.ds(c * N + r, B)], sems[k])
        for i in range(min(nchunks, NBUF)):
            pltpu.make_async_copy(bufs[i], sums.at[pl.ds(0, B)],
                                  sems[i]).wait()

    @pl.when(s < NS - 1)
    def _():
        drain(CROWS // B)

    @pl.when(s == NS - 1)
    def _():
        drain((N - (NS - 1) * CROWS) // B)


_agg = pl.kernel(
    _agg_body,
    out_type=jax.ShapeDtypeStruct((2 * N, D), jnp.float32),
    mesh=_MESH,
    scratch_types=[
        pltpu.VMEM((IB, B), jnp.int32),
        pltpu.VMEM((IB, B), jnp.int32),
        pltpu.VMEM((B, D), jnp.float32),
        pltpu.VMEM((B, D), jnp.float32),
        pltpu.VMEM((B, D), jnp.float32),
        pltpu.VMEM_SHARED((NPAD, D), jnp.float32),
        pltpu.SemaphoreType.DMA,
        pltpu.SemaphoreType.DMA,
        pltpu.SemaphoreType.DMA,
    ],
)


def _cnt_body(dst_r, zeros, ones, cnt_out, dstv, zb, onesb, cacc):
    c = lax.axis_index("c")
    s = lax.axis_index("s")
    wid = c * NS + s

    pltpu.sync_copy(dst_r.at[wid], dstv)
    pltpu.sync_copy(zeros, zb)
    pltpu.sync_copy(ones, onesb)
    base = s * CROWS
    for i in range(CROWS // B):
        pltpu.sync_copy(zb, cacc.at[pl.ds(base + i * B, B)])
    plsc.subcore_barrier()

    def scat(j, carry):
        pltpu.sync_copy(onesb, cacc.at[dstv.at[j]], add=True)
        return carry

    lax.fori_loop(0, CB, scat, 0)
    plsc.subcore_barrier()

    for i in range(CROWS // B):
        r = base + i * B
        pltpu.sync_copy(cacc.at[pl.ds(r, B)], zb)
        pltpu.sync_copy(zb, cnt_out.at[c, pl.ds(r, B)])


_cnt = pl.kernel(
    _cnt_body,
    out_type=jax.ShapeDtypeStruct((NC, NPAD, D), jnp.float32),
    mesh=_MESH,
    scratch_types=[
        pltpu.VMEM((CB, B), jnp.int32),
        pltpu.VMEM((B, D), jnp.float32),
        pltpu.VMEM((B, D), jnp.float32),
        pltpu.VMEM_SHARED((NPAD, D), jnp.float32),
    ],
)


def _layer_tc(s2n, cnta, cntb, W, b, relu):
    BR = 400

    def body(x0_ref, x1_ref, ca_ref, cb_ref, w_ref, b_ref, o_ref):
        cnt = ca_ref[...] + cb_ref[...]
        r = 1.0 / jnp.maximum(cnt, 1.0)
        x0 = x0_ref[...] * r
        x1 = x1_ref[...] * r
        h = jnp.dot(x0, w_ref[0:D, :], preferred_element_type=jnp.float32)
        h = h + jnp.dot(x1, w_ref[D:H, :], preferred_element_type=jnp.float32)
        h = h + b_ref[...]
        if relu:
            h = jnp.maximum(h, 0.0)
        o_ref[...] = h

    nb = N // BR
    return pl.pallas_call(
        body,
        grid=(2, nb),
        in_specs=[
            pl.BlockSpec((BR, D), lambda g, i: (i, 0)),
            pl.BlockSpec((BR, D), lambda g, i: (i + nb, 0)),
            pl.BlockSpec((BR, 1), lambda g, i: (i, 0)),
            pl.BlockSpec((BR, 1), lambda g, i: (i, 0)),
            pl.BlockSpec((H, D), lambda g, i: (0, g)),
            pl.BlockSpec((1, D), lambda g, i: (0, g)),
        ],
        out_specs=pl.BlockSpec((BR, D), lambda g, i: (g * nb + i, 0)),
        out_shape=jax.ShapeDtypeStruct((2 * N, D), jnp.float32),
    )(s2n, s2n, cnta, cntb, W, b.reshape(1, H))


def _head_tc(hi, hs, Wpi, bpi, Wps, bps, Wf, bf):
    BR = 400
    nb = N // BR

    def body(hi0_ref, hi1_ref, hs0_ref, hs1_ref,
             wpi_ref, bpi_ref, wps_ref, bps_ref, wf_ref, bf_ref,
             h_ref, zi_ref, zs_ref):
        hi0 = hi0_ref[...]
        hi1 = hi1_ref[...]
        hs0 = hs0_ref[...]
        hs1 = hs1_ref[...]
        zi = jnp.dot(hi0, wpi_ref[0:D, :], preferred_element_type=jnp.float32)
        zi = zi + jnp.dot(hi1, wpi_ref[D:H, :], preferred_element_type=jnp.float32)
        zi_ref[...] = jnp.maximum(zi + bpi_ref[...], 0.0)
        zs = jnp.dot(hs0, wps_ref[0:D, :], preferred_element_type=jnp.float32)
        zs = zs + jnp.dot(hs1, wps_ref[D:H, :], preferred_element_type=jnp.float32)
        zs_ref[...] = jnp.maximum(zs + bps_ref[...], 0.0)
        h = jnp.dot(hi0 + hs0, wf_ref[0:D, :], preferred_element_type=jnp.float32)
        h = h + jnp.dot(hi1 + hs1, wf_ref[D:H, :], preferred_element_type=jnp.float32)
        h_ref[...] = h + bf_ref[...]

    return pl.pallas_call(
        body,
        grid=(nb,),
        in_specs=[
            pl.BlockSpec((BR, D), lambda i: (i, 0)),
            pl.BlockSpec((BR, D), lambda i: (i + nb, 0)),
            pl.BlockSpec((BR, D), lambda i: (i, 0)),
            pl.BlockSpec((BR, D), lambda i: (i + nb, 0)),
            pl.BlockSpec((H, H), lambda i: (0, 0)),
            pl.BlockSpec((1, H), lambda i: (0, 0)),
            pl.BlockSpec((H, H), lambda i: (0, 0)),
            pl.BlockSpec((1, H), lambda i: (0, 0)),
            pl.BlockSpec((H, D), lambda i: (0, 0)),
            pl.BlockSpec((1, D), lambda i: (0, 0)),
        ],
        out_specs=[
            pl.BlockSpec((BR, D), lambda i: (i, 0)),
            pl.BlockSpec((BR, H), lambda i: (i, 0)),
            pl.BlockSpec((BR, H), lambda i: (i, 0)),
        ],
        out_shape=[
            jax.ShapeDtypeStruct((N, D), jnp.float32),
            jax.ShapeDtypeStruct((N, H), jnp.float32),
            jax.ShapeDtypeStruct((N, H), jnp.float32),
        ],
    )(hi, hi, hs, hs, Wpi, bpi.reshape(1, H), Wps, bps.reshape(1, H),
      Wf, bf.reshape(1, D))


def kernel(x, x_sim, edge_index_inter, edge_index_sim,
           W1i, b1i, W2i, b2i, W3i, b3i,
           W1s, b1s, W2s, b2s, W3s, b3s,
           Wpi, bpi, Wps, bps, Wf, bf):
    x2n = jnp.concatenate([x, x_sim], axis=0)
    zeros = jnp.zeros((B, D), jnp.float32)
    ones = jnp.ones((B, D), jnp.float32)

    def prep(edge_index):
        src = edge_index[0].reshape(NS, NIB, IB, B)
        dst = edge_index[1].reshape(NS, NIB, IB, B)
        src_r = jnp.stack([src, src + N])
        dst_c = edge_index[1].reshape(NC * NS, CB, B)
        cnt = _cnt(dst_c, zeros, ones)
        return src_r, dst, cnt[0, :N, 0:1], cnt[1, :N, 0:1]

    gi = prep(edge_index_inter)
    gs = prep(edge_index_sim)
    Wi = ((W1i, b1i), (W2i, b2i), (W3i, b3i))
    Ws = ((W1s, b1s), (W2s, b2s), (W3s, b3s))
    hi, hs = x2n, x2n
    for l in range(3):
        relu = l < 2
        si = _agg(hi, gi[0], gi[1], zeros)
        ss = _agg(hs, gs[0], gs[1], zeros)
        hi = _layer_tc(si, gi[2], gi[3], Wi[l][0], Wi[l][1], relu=relu)
        hs = _layer_tc(ss, gs[2], gs[3], Ws[l][0], Ws[l][1], relu=relu)
    return _head_tc(hi, hs, Wpi, bpi, Wps, bps, Wf, bf)


# speedup vs baseline: 1.4266x; 1.0281x over previous
import jax
import jax.numpy as jnp
from jax import lax
from jax.experimental import pallas as pl
from jax.experimental.pallas import tpu as pltpu
from jax.experimental.pallas import tpu_sc as plsc

N = 10000
D = 128
H = 256
E = 320000

NC = 2
NS = 16
B = 80
NB = E // NS // B
IB = 25
NIB = NB // IB
NBUF = 3
NPAD = 10240
CROWS = NPAD // NS
CB = 125

_MESH = plsc.VectorSubcoreMesh(core_axis_name="c", subcore_axis_name="s")


def _agg_body(x2n, src_r, dst_r, zeros, sums,
              srcv, dstv, buf0, buf1, buf2, acc, sem0, sem1, sem2):
    c = lax.axis_index("c")
    s = lax.axis_index("s")

    pltpu.sync_copy(zeros, buf0)
    base = s * CROWS
    for i in range(CROWS // B):
        pltpu.async_copy(buf0, acc.at[pl.ds(base + i * B, B)], sem0)
    for i in range(CROWS // B):
        pltpu.make_async_copy(buf0, acc.at[pl.ds(base, B)], sem0).wait()
    plsc.subcore_barrier()

    bufs = (buf0, buf1, buf2)
    sems = (sem0, sem1, sem2)

    def gstart(j, k):
        pltpu.async_copy(x2n.at[srcv.at[j]], bufs[k], sems[k])

    def gwait(k):
        pltpu.make_async_copy(x2n.at[pl.ds(0, B)], bufs[k], sems[k]).wait()

    def scat(j, k):
        pltpu.sync_copy(bufs[k], acc.at[dstv.at[j]], add=True)

    def chunk(g, carry):
        pltpu.sync_copy(src_r.at[c, s, g], srcv)
        pltpu.sync_copy(dst_r.at[s, g], dstv)
        for k in range(NBUF):
            gstart(k, k)

        def step(t, cy):
            j = NBUF * t
            for k in range(NBUF):
                gwait(k)
                scat(j + k, k)
                gstart(j + NBUF + k, k)
            return cy

        lax.fori_loop(0, (IB - 1) // NBUF - 1, step, 0)
        gwait(0)
        scat(IB - 4, 0)
        gstart(IB - 1, 0)
        gwait(1)
        scat(IB - 3, 1)
        gwait(2)
        scat(IB - 2, 2)
        gwait(0)
        scat(IB - 1, 0)
        return carry

    lax.fori_loop(0, NIB, chunk, 0)
    plsc.subcore_barrier()

    def drain(nchunks):
        for i in range(nchunks):
            r = base + i * B
            k = i % NBUF
            if i >= NBUF:
                pltpu.make_async_copy(bufs[k], sums.at[pl.ds(0, B)],
                                      sems[k]).wait()
            pltpu.sync_copy(acc.at[pl.ds(r, B)], bufs[k])
            pltpu.async_copy(bufs[k], sums.at[pl.ds(c * N + r, B)], sems[k])
        for i in range(min(nchunks, NBUF)):
            pltpu.make_async_copy(bufs[i], sums.at[pl.ds(0, B)],
                                  sems[i]).wait()

    @pl.when(s < NS - 1)
    def _():
        drain(CROWS // B)

    @pl.when(s == NS - 1)
    def _():
        drain((N - (NS - 1) * CROWS) // B)


_agg = pl.kernel(
    _agg_body,
    out_type=jax.ShapeDtypeStruct((2 * N, D), jnp.float32),
    mesh=_MESH,
    scratch_types=[
        pltpu.VMEM((IB, B), jnp.int32),
        pltpu.VMEM((IB, B), jnp.int32),
        pltpu.VMEM((B, D), jnp.float32),
        pltpu.VMEM((B, D), jnp.float32),
        pltpu.VMEM((B, D), jnp.float32),
        pltpu.VMEM_SHARED((NPAD, D), jnp.float32),
        pltpu.SemaphoreType.DMA,
        pltpu.SemaphoreType.DMA,
        pltpu.SemaphoreType.DMA,
    ],
)


def _cnt_body(dst_r, zeros, ones, cnt_out, dstv, zb, onesb, cacc):
    c = lax.axis_index("c")
    s = lax.axis_index("s")
    wid = c * NS + s

    pltpu.sync_copy(dst_r.at[wid], dstv)
    pltpu.sync_copy(zeros, zb)
    pltpu.sync_copy(ones, onesb)
    base = s * CROWS
    for i in range(CROWS // B):
        pltpu.sync_copy(zb, cacc.at[pl.ds(base + i * B, B)])
    plsc.subcore_barrier()

    def scat(j, carry):
        pltpu.sync_copy(onesb, cacc.at[dstv.at[j]], add=True)
        return carry

    lax.fori_loop(0, CB, scat, 0)
    plsc.subcore_barrier()

    for i in range(CROWS // B):
        r = base + i * B
        pltpu.sync_copy(cacc.at[pl.ds(r, B)], zb)
        pltpu.sync_copy(zb, cnt_out.at[c, pl.ds(r, B)])


_cnt = pl.kernel(
    _cnt_body,
    out_type=jax.ShapeDtypeStruct((NC, NPAD, D), jnp.float32),
    mesh=_MESH,
    scratch_types=[
        pltpu.VMEM((CB, B), jnp.int32),
        pltpu.VMEM((B, D), jnp.float32),
        pltpu.VMEM((B, D), jnp.float32),
        pltpu.VMEM_SHARED((NPAD, D), jnp.float32),
    ],
)


def _layer_tc(s2n, cnta, cntb, W, b, relu):
    BR = 400

    def body(x0_ref, x1_ref, ca_ref, cb_ref, w_ref, b_ref, o_ref):
        cnt = ca_ref[...] + cb_ref[...]
        r = 1.0 / jnp.maximum(cnt, 1.0)
        x0 = x0_ref[...] * r
        x1 = x1_ref[...] * r
        h = jnp.dot(x0, w_ref[0:D, :], preferred_element_type=jnp.float32)
        h = h + jnp.dot(x1, w_ref[D:H, :], preferred_element_type=jnp.float32)
        h = h + b_ref[...]
        if relu:
            h = jnp.maximum(h, 0.0)
        o_ref[...] = h

    nb = N // BR
    return pl.pallas_call(
        body,
        grid=(2, nb),
        in_specs=[
            pl.BlockSpec((BR, D), lambda g, i: (i, 0)),
            pl.BlockSpec((BR, D), lambda g, i: (i + nb, 0)),
            pl.BlockSpec((BR, 1), lambda g, i: (i, 0)),
            pl.BlockSpec((BR, 1), lambda g, i: (i, 0)),
            pl.BlockSpec((H, D), lambda g, i: (0, g)),
            pl.BlockSpec((1, D), lambda g, i: (0, g)),
        ],
        out_specs=pl.BlockSpec((BR, D), lambda g, i: (g * nb + i, 0)),
        out_shape=jax.ShapeDtypeStruct((2 * N, D), jnp.float32),
    )(s2n, s2n, cnta, cntb, W, b.reshape(1, H))


def _tail_tc(si, ss, cia, cib, csa, csb, W3i, b3i, W3s, b3s,
             Wpi, bpi, Wps, bps, Wf, bf):
    BR = 400
    nb = N // BR

    def _mean_dot(x0_ref, x1_ref, ca_ref, cb_ref, w_ref, b_ref):
        cnt = ca_ref[...] + cb_ref[...]
        r = 1.0 / jnp.maximum(cnt, 1.0)
        x0 = x0_ref[...] * r
        x1 = x1_ref[...] * r
        h = jnp.dot(x0, w_ref[0:D, :], preferred_element_type=jnp.float32)
        h = h + jnp.dot(x1, w_ref[D:H, :], preferred_element_type=jnp.float32)
        return h + b_ref[...]

    def body(si0_ref, si1_ref, ss0_ref, ss1_ref,
             cia_ref, cib_ref, csa_ref, csb_ref,
             w3i_ref, b3i_ref, w3s_ref, b3s_ref,
             wpi_ref, bpi_ref, wps_ref, bps_ref, wf_ref, bf_ref,
             h_ref, zi_ref, zs_ref):
        hi = _mean_dot(si0_ref, si1_ref, cia_ref, cib_ref, w3i_ref, b3i_ref)
        hs = _mean_dot(ss0_ref, ss1_ref, csa_ref, csb_ref, w3s_ref, b3s_ref)
        zi = jnp.dot(hi, wpi_ref[...], preferred_element_type=jnp.float32)
        zi_ref[...] = jnp.maximum(zi + bpi_ref[...], 0.0)
        zs = jnp.dot(hs, wps_ref[...], preferred_element_type=jnp.float32)
        zs_ref[...] = jnp.maximum(zs + bps_ref[...], 0.0)
        h = jnp.dot(hi + hs, wf_ref[...], preferred_element_type=jnp.float32)
        h_ref[...] = h + bf_ref[...]

    full = lambda shape: pl.BlockSpec(shape, lambda i: tuple(0 for _ in shape))
    return pl.pallas_call(
        body,
        grid=(nb,),
        in_specs=[
            pl.BlockSpec((BR, D), lambda i: (i, 0)),
            pl.BlockSpec((BR, D), lambda i: (i + nb, 0)),
            pl.BlockSpec((BR, D), lambda i: (i, 0)),
            pl.BlockSpec((BR, D), lambda i: (i + nb, 0)),
            pl.BlockSpec((BR, 1), lambda i: (i, 0)),
            pl.BlockSpec((BR, 1), lambda i: (i, 0)),
            pl.BlockSpec((BR, 1), lambda i: (i, 0)),
            pl.BlockSpec((BR, 1), lambda i: (i, 0)),
            full((H, H)), full((1, H)), full((H, H)), full((1, H)),
            full((H, H)), full((1, H)), full((H, H)), full((1, H)),
            full((H, D)), full((1, D)),
        ],
        out_specs=[
            pl.BlockSpec((BR, D), lambda i: (i, 0)),
            pl.BlockSpec((BR, H), lambda i: (i, 0)),
            pl.BlockSpec((BR, H), lambda i: (i, 0)),
        ],
        out_shape=[
            jax.ShapeDtypeStruct((N, D), jnp.float32),
            jax.ShapeDtypeStruct((N, H), jnp.float32),
            jax.ShapeDtypeStruct((N, H), jnp.float32),
        ],
    )(si, si, ss, ss, cia, cib, csa, csb,
      W3i, b3i.reshape(1, H), W3s, b3s.reshape(1, H),
      Wpi, bpi.reshape(1, H), Wps, bps.reshape(1, H),
      Wf, bf.reshape(1, D))


def kernel(x, x_sim, edge_index_inter, edge_index_sim,
           W1i, b1i, W2i, b2i, W3i, b3i,
           W1s, b1s, W2s, b2s, W3s, b3s,
           Wpi, bpi, Wps, bps, Wf, bf):
    x2n = jnp.concatenate([x, x_sim], axis=0)
    zeros = jnp.zeros((B, D), jnp.float32)
    ones = jnp.ones((B, D), jnp.float32)

    def prep(edge_index):
        src = edge_index[0].reshape(NS, NIB, IB, B)
        dst = edge_index[1].reshape(NS, NIB, IB, B)
        src_r = jnp.stack([src, src + N])
        dst_c = edge_index[1].reshape(NC * NS, CB, B)
        cnt = _cnt(dst_c, zeros, ones)
        return src_r, dst, cnt[0, :N, 0:1], cnt[1, :N, 0:1]

    gi = prep(edge_index_inter)
    gs = prep(edge_index_sim)
    Wi = ((W1i, b1i), (W2i, b2i), (W3i, b3i))
    Ws = ((W1s, b1s), (W2s, b2s), (W3s, b3s))
    hi, hs = x2n, x2n
    for l in range(2):
        si = _agg(hi, gi[0], gi[1], zeros)
        ss = _agg(hs, gs[0], gs[1], zeros)
        hi = _layer_tc(si, gi[2], gi[3], Wi[l][0], Wi[l][1], relu=True)
        hs = _layer_tc(ss, gs[2], gs[3], Ws[l][0], Ws[l][1], relu=True)
    si = _agg(hi, gi[0], gi[1], zeros)
    ss = _agg(hs, gs[0], gs[1], zeros)
    return _tail_tc(si, ss, gi[2], gi[3], gs[2], gs[3],
                    W3i, b3i, W3s, b3s, Wpi, bpi, Wps, bps, Wf, bf)
